# Initial kernel scaffold; baseline (speedup 1.0000x reference)
import jax
import jax.numpy as jnp
from jax.experimental import pallas as pl


def kernel(hidden_states, gate_W, W_gate, W_up, W_down, ln_g, ln_b):
    raise NotImplementedError("write your pallas kernel here")


# speedup vs baseline: 1.2979x; 1.2979x over previous
import functools

import jax
import jax.numpy as jnp
from jax.experimental import pallas as pl
from jax.experimental.pallas import tpu as pltpu

H = 768
F = 3072
E = 8
K = 2
T = 2048
A = T * K
TILE = 512
NT = (A + E * TILE) // TILE
A_PAD = NT * TILE
FC = 1536
NF = F // FC
RT = 256
AUX_COEF = 0.01


def _router_body(x_ref, g_ref, b_ref, gw_ref,
                 normed_ref, isel_ref, wsel_ref, psum_ref, csum_ref, aux_ref):
    i = pl.program_id(0)
    nsteps = pl.num_programs(0)

    @pl.when(i == 0)
    def _init():
        psum_ref[...] = jnp.zeros_like(psum_ref)
        csum_ref[...] = jnp.zeros_like(csum_ref)

    x = x_ref[...]
    mu = jnp.mean(x, axis=-1, keepdims=True)
    xc = x - mu
    var = jnp.mean(xc * xc, axis=-1, keepdims=True)
    normed = xc * jax.lax.rsqrt(var + 1e-5) * g_ref[...] + b_ref[...]
    normed_ref[...] = normed

    logits = jax.lax.dot_general(normed, gw_ref[...],
                                 (((1,), (1,)), ((), ())),
                                 preferred_element_type=jnp.float32)
    eidx = jax.lax.broadcasted_iota(jnp.int32, logits.shape, 1)
    m1 = jnp.max(logits, axis=-1, keepdims=True)
    i1 = jnp.min(jnp.where(logits == m1, eidx, E), axis=-1, keepdims=True)
    logits2 = jnp.where(eidx == i1, -jnp.inf, logits)
    m2 = jnp.max(logits2, axis=-1, keepdims=True)
    i2 = jnp.min(jnp.where(logits2 == m2, eidx, E), axis=-1, keepdims=True)
    e2 = jnp.exp(m2 - m1)
    w1 = 1.0 / (1.0 + e2)
    w2 = e2 * w1
    isel_ref[...] = jnp.concatenate([i1, i2], axis=1)
    wsel_ref[...] = jnp.concatenate([w1, w2], axis=1)

    ex = jnp.exp(logits - m1)
    probs = ex / jnp.sum(ex, axis=-1, keepdims=True)
    psum_ref[...] += jnp.sum(probs, axis=0, keepdims=True)
    sel = jnp.logical_or(eidx == i1, eidx == i2).astype(jnp.float32)
    csum_ref[...] += jnp.sum(sel, axis=0, keepdims=True)

    @pl.when(i == nsteps - 1)
    def _fin():
        frac = csum_ref[...] / T
        pmean = psum_ref[...] / T
        aux_ref[...] = (AUX_COEF * E * jnp.sum(frac * pmean)).reshape(1, 1)


def _router(x, gate_W, ln_g, ln_b):
    return pl.pallas_call(
        _router_body,
        grid=(T // RT,),
        in_specs=[
            pl.BlockSpec((RT, H), lambda i: (i, 0)),
            pl.BlockSpec((1, H), lambda i: (0, 0)),
            pl.BlockSpec((1, H), lambda i: (0, 0)),
            pl.BlockSpec((E, H), lambda i: (0, 0)),
        ],
        out_specs=[
            pl.BlockSpec((RT, H), lambda i: (i, 0)),
            pl.BlockSpec((RT, K), lambda i: (i, 0)),
            pl.BlockSpec((RT, K), lambda i: (i, 0)),
            pl.BlockSpec((1, E), lambda i: (0, 0)),
            pl.BlockSpec((1, E), lambda i: (0, 0)),
            pl.BlockSpec((1, 1), lambda i: (0, 0)),
        ],
        out_shape=[
            jax.ShapeDtypeStruct((T, H), jnp.float32),
            jax.ShapeDtypeStruct((T, K), jnp.int32),
            jax.ShapeDtypeStruct((T, K), jnp.float32),
            jax.ShapeDtypeStruct((1, E), jnp.float32),
            jax.ShapeDtypeStruct((1, E), jnp.float32),
            jax.ShapeDtypeStruct((1, 1), jnp.float32),
        ],
        compiler_params=pltpu.CompilerParams(
            dimension_semantics=("arbitrary",)),
    )(x, ln_g.reshape(1, H), ln_b.reshape(1, H), gate_W)


def _ffn_body(te_ref, tv_ref, xs_ref, ws_ref, wg_ref, wu_ref, wd_ref, y_ref):
    i = pl.program_id(0)
    j = pl.program_id(1)

    @pl.when(tv_ref[i] == 1)
    def _():
        x = xs_ref[...]
        g = jnp.dot(x, wg_ref[0], preferred_element_type=jnp.float32)
        u = jnp.dot(x, wu_ref[0], preferred_element_type=jnp.float32)
        h = (g * jax.nn.sigmoid(g)) * u * ws_ref[...]
        part = jnp.dot(h, wd_ref[0], preferred_element_type=jnp.float32)

        @pl.when(j == 0)
        def _a():
            y_ref[...] = part

        @pl.when(j > 0)
        def _b():
            y_ref[...] += part


def _grouped_ffn(xs, ws, W_gate, W_up, W_down, tile_expert, tile_valid):
    grid_spec = pltpu.PrefetchScalarGridSpec(
        num_scalar_prefetch=2,
        grid=(NT, NF),
        in_specs=[
            pl.BlockSpec((TILE, H), lambda i, j, te, tv: (i, 0)),
            pl.BlockSpec((TILE, 1), lambda i, j, te, tv: (i, 0)),
            pl.BlockSpec((1, H, FC), lambda i, j, te, tv: (te[i], 0, j)),
            pl.BlockSpec((1, H, FC), lambda i, j, te, tv: (te[i], 0, j)),
            pl.BlockSpec((1, FC, H), lambda i, j, te, tv: (te[i], j, 0)),
        ],
        out_specs=pl.BlockSpec((TILE, H), lambda i, j, te, tv: (i, 0)),
    )
    return pl.pallas_call(
        _ffn_body,
        grid_spec=grid_spec,
        out_shape=jax.ShapeDtypeStruct((A_PAD, H), jnp.float32),
        compiler_params=pltpu.CompilerParams(
            dimension_semantics=("arbitrary", "arbitrary")),
    )(tile_expert, tile_valid, xs, ws, W_gate, W_up, W_down)


def _combine_body(x_ref, y0_ref, y1_ref, out_ref):
    out_ref[...] = x_ref[...] + y0_ref[...] + y1_ref[...]


def _combine(x, y0, y1):
    return pl.pallas_call(
        _combine_body,
        grid=(T // RT,),
        in_specs=[pl.BlockSpec((RT, H), lambda i: (i, 0))] * 3,
        out_specs=pl.BlockSpec((RT, H), lambda i: (i, 0)),
        out_shape=jax.ShapeDtypeStruct((T, H), jnp.float32),
    )(x, y0, y1)


def kernel(hidden_states, gate_W, W_gate, W_up, W_down, ln_g, ln_b):
    B, S, _ = hidden_states.shape
    x = hidden_states.reshape(T, H)

    normed, isel, wsel, _, _, aux = _router(x, gate_W, ln_g, ln_b)

    flat_e = isel.reshape(A)
    oh = (flat_e[:, None] == jnp.arange(E, dtype=jnp.int32)[None, :])
    pre = jnp.cumsum(oh.astype(jnp.int32), axis=0)
    counts = pre[-1]
    rank = jnp.take_along_axis(pre, flat_e[:, None], axis=1)[:, 0] - 1
    tiles_per = (counts + TILE - 1) // TILE
    cum_tiles = jnp.cumsum(tiles_per)
    group_start = (cum_tiles - tiles_per) * TILE
    pos = group_start[flat_e] + rank
    tok = jnp.arange(A, dtype=jnp.int32) // K
    tok_sorted = jnp.zeros((A_PAD,), jnp.int32).at[pos].set(tok)
    w_sorted = jnp.zeros((A_PAD,), jnp.float32).at[pos].set(wsel.reshape(A))
    tidx = jnp.arange(NT, dtype=jnp.int32)
    tile_expert = jnp.minimum(
        jnp.sum((tidx[:, None] >= cum_tiles[None, :]).astype(jnp.int32), axis=1),
        E - 1).astype(jnp.int32)
    tile_valid = (tidx < cum_tiles[-1]).astype(jnp.int32)

    xs = jnp.take(normed, tok_sorted, axis=0)

    y = _grouped_ffn(xs, w_sorted.reshape(A_PAD, 1),
                     W_gate, W_up, W_down, tile_expert, tile_valid)

    p01 = pos.reshape(T, K)
    y0 = jnp.take(y, p01[:, 0], axis=0)
    y1 = jnp.take(y, p01[:, 1], axis=0)
    out = _combine(x, y0, y1)

    return out.reshape(B, S, H), aux[0, 0]
